# Initial kernel scaffold; baseline (speedup 1.0000x reference)
import jax
import jax.numpy as jnp
from jax.experimental import pallas as pl


def kernel(x, edge_index, edge_attr, batch, params):
    raise NotImplementedError("write your pallas kernel here")


# speedup vs baseline: 1.5216x; 1.5216x over previous
import functools
import jax
import jax.numpy as jnp
from jax import lax
from jax.experimental import pallas as pl
from jax.experimental.pallas import tpu as pltpu
from jax.experimental.pallas import tpu_sc as plsc

N_NODES = 50000
N_EDGES = 800000
IN_CH = 128
EDGE_DIM = 16
HIDDEN = 64
HH = 32
NUM_GRAPHS = 128

NC = 2
NS = 16
PER_TILE = N_EDGES // NS
CH = 80
NCHUNK = PER_TILE // CH
ROWS_A = 3128
ROWS_LAST = N_NODES - (NS - 1) * ROWS_A


def _sc_pass_body(h0, h1, ea0, ea1, src, dst, zrows,
                  agg0, agg1, acc, sidx, didx, gbuf, eabuf, sem):
  c = lax.axis_index("c")
  s = lax.axis_index("s")

  def do_half(h_ref, ea_ref, agg_ref):
    @pl.when(s < NS - 1)
    def _():
      pltpu.sync_copy(zrows.at[pl.ds(0, ROWS_A)], acc.at[pl.ds(s * ROWS_A, ROWS_A)])

    @pl.when(s == NS - 1)
    def _():
      pltpu.sync_copy(zrows.at[pl.ds(0, ROWS_LAST)],
                      acc.at[pl.ds((NS - 1) * ROWS_A, ROWS_LAST)])

    plsc.subcore_barrier()

    base = s * PER_TILE

    @pl.loop(0, NCHUNK)
    def _chunk(i):
      e0 = base + i * CH
      pltpu.sync_copy(src.at[pl.ds(e0, CH)], sidx)
      pltpu.sync_copy(dst.at[pl.ds(e0, CH)], didx)
      pltpu.sync_copy(ea_ref.at[pl.ds(e0, CH)], eabuf)
      pltpu.async_copy(h_ref.at[sidx], gbuf, sem).wait()

      @pl.loop(0, CH, unroll=4)
      def _row(r):
        a = gbuf[r, pl.ds(0, 16)] + eabuf[r, pl.ds(0, 16)]
        gbuf[r, pl.ds(0, 16)] = jnp.maximum(a, 0.0)
        b = gbuf[r, pl.ds(16, 16)] + eabuf[r, pl.ds(16, 16)]
        gbuf[r, pl.ds(16, 16)] = jnp.maximum(b, 0.0)

      pltpu.sync_copy(gbuf, acc.at[didx], add=True)

    plsc.subcore_barrier()

    @pl.when(s < NS - 1)
    def _():
      pltpu.sync_copy(acc.at[pl.ds(s * ROWS_A, ROWS_A)],
                      agg_ref.at[pl.ds(s * ROWS_A, ROWS_A)])

    @pl.when(s == NS - 1)
    def _():
      pltpu.sync_copy(acc.at[pl.ds((NS - 1) * ROWS_A, ROWS_LAST)],
                      agg_ref.at[pl.ds((NS - 1) * ROWS_A, ROWS_LAST)])

  @pl.when(c == 0)
  def _():
    do_half(h0, ea0, agg0)

  @pl.when(c == 1)
  def _():
    do_half(h1, ea1, agg1)


@functools.cache
def _get_sc_pass():
  return pl.kernel(
      _sc_pass_body,
      out_type=(jax.ShapeDtypeStruct((N_NODES, HH), jnp.float32),
                jax.ShapeDtypeStruct((N_NODES, HH), jnp.float32)),
      mesh=plsc.VectorSubcoreMesh(core_axis_name="c", subcore_axis_name="s",
                                  num_cores=NC, num_subcores=NS),
      compiler_params=pltpu.CompilerParams(use_tc_tiling_on_sc=False),
      scratch_types=[
          pltpu.VMEM_SHARED((N_NODES, HH), jnp.float32),
          pltpu.VMEM((CH,), jnp.int32),
          pltpu.VMEM((CH,), jnp.int32),
          pltpu.VMEM((CH, HH), jnp.float32),
          pltpu.VMEM((CH, HH), jnp.float32),
          pltpu.SemaphoreType.DMA,
      ],
  )


def _sc_pass(*args):
  return _get_sc_pass()(*args)


BN_ENC = 2000
BE = 8000
BM = 2000


def _enc_body(x_ref, w_ref, b_ref, h0_ref, h1_ref):
  h = jnp.dot(x_ref[...], w_ref[...], preferred_element_type=jnp.float32)
  h = h + b_ref[...]
  h0_ref[...] = h[:, :HH]
  h1_ref[...] = h[:, HH:]


def _enc(x, w, b):
  return pl.pallas_call(
      _enc_body,
      grid=(N_NODES // BN_ENC,),
      in_specs=[
          pl.BlockSpec((BN_ENC, IN_CH), lambda i: (i, 0)),
          pl.BlockSpec((IN_CH, HIDDEN), lambda i: (0, 0)),
          pl.BlockSpec((1, HIDDEN), lambda i: (0, 0)),
      ],
      out_specs=[
          pl.BlockSpec((BN_ENC, HH), lambda i: (i, 0)),
          pl.BlockSpec((BN_ENC, HH), lambda i: (i, 0)),
      ],
      out_shape=[
          jax.ShapeDtypeStruct((N_NODES, HH), jnp.float32),
          jax.ShapeDtypeStruct((N_NODES, HH), jnp.float32),
      ],
  )(x, w, b)


def _ea_body(e_ref, w_ref, b_ref, o0_ref, o1_ref):
  o = jnp.dot(e_ref[...], w_ref[...], preferred_element_type=jnp.float32)
  o = o + b_ref[...]
  o0_ref[...] = o[:, :HH]
  o1_ref[...] = o[:, HH:]


def _ea_embed(edge_attr, w, b):
  return pl.pallas_call(
      _ea_body,
      grid=(N_EDGES // BE,),
      in_specs=[
          pl.BlockSpec((BE, EDGE_DIM), lambda i: (i, 0)),
          pl.BlockSpec((EDGE_DIM, HIDDEN), lambda i: (0, 0)),
          pl.BlockSpec((1, HIDDEN), lambda i: (0, 0)),
      ],
      out_specs=[
          pl.BlockSpec((BE, HH), lambda i: (i, 0)),
          pl.BlockSpec((BE, HH), lambda i: (i, 0)),
      ],
      out_shape=[
          jax.ShapeDtypeStruct((N_EDGES, HH), jnp.float32),
          jax.ShapeDtypeStruct((N_EDGES, HH), jnp.float32),
      ],
  )(edge_attr, w, b)


def _onehot(batch_row):
  gid = lax.broadcasted_iota(jnp.int32, (1, NUM_GRAPHS), 1)
  return (batch_row[:, None] == gid).astype(jnp.float32)


def _m0_body(h0, h1, a0, a1, bt, eps_ref,
             w1l, b1l, w2l, b2l, w1g, b1g, w2g, b2g,
             hl0, hl1, hg0, hg1, pool):
  i = pl.program_id(0)
  h = jnp.concatenate([h0[...], h1[...]], axis=1)
  agg = jnp.concatenate([a0[...], a1[...]], axis=1)
  oh = _onehot(bt[0, 0, :])

  pcs = []
  for (k, w1, b1, w2, b2, ho0, ho1) in (
      (0, w1l, b1l, w2l, b2l, hl0, hl1),
      (1, w1g, b1g, w2g, b2g, hg0, hg1)):
    z = (1.0 + eps_ref[k]) * h + agg
    t = jnp.maximum(jnp.dot(z, w1[...], preferred_element_type=jnp.float32)
                    + b1[...], 0.0)
    z2 = jnp.dot(t, w2[...], preferred_element_type=jnp.float32) + b2[...]
    hn = jnp.maximum(z2, 0.0)
    ho0[...] = hn[:, :HH]
    ho1[...] = hn[:, HH:]
    pcs.append(lax.dot_general(oh, hn, (((0,), (0,)), ((), ())),
                               precision=lax.Precision.HIGHEST,
                               preferred_element_type=jnp.float32))

  pc = jnp.concatenate(pcs, axis=1)

  @pl.when(i == 0)
  def _():
    pool[...] = pc

  @pl.when(i > 0)
  def _():
    pool[...] += pc


def _mlp0(h0, h1, a0, a1, batch3, eps2, wl, wg):
  wspec = [
      pl.BlockSpec((HIDDEN, HIDDEN), lambda i: (0, 0)),
      pl.BlockSpec((1, HIDDEN), lambda i: (0, 0)),
      pl.BlockSpec((HIDDEN, HIDDEN), lambda i: (0, 0)),
      pl.BlockSpec((1, HIDDEN), lambda i: (0, 0)),
  ]
  return pl.pallas_call(
      _m0_body,
      grid=(N_NODES // BM,),
      in_specs=[
          pl.BlockSpec((BM, HH), lambda i: (i, 0)),
          pl.BlockSpec((BM, HH), lambda i: (i, 0)),
          pl.BlockSpec((BM, HH), lambda i: (i, 0)),
          pl.BlockSpec((BM, HH), lambda i: (i, 0)),
          pl.BlockSpec((1, 1, BM), lambda i: (i, 0, 0)),
          pl.BlockSpec(memory_space=pltpu.SMEM),
      ] + wspec + wspec,
      out_specs=[
          pl.BlockSpec((BM, HH), lambda i: (i, 0)),
          pl.BlockSpec((BM, HH), lambda i: (i, 0)),
          pl.BlockSpec((BM, HH), lambda i: (i, 0)),
          pl.BlockSpec((BM, HH), lambda i: (i, 0)),
          pl.BlockSpec((NUM_GRAPHS, 2 * HIDDEN), lambda i: (0, 0)),
      ],
      out_shape=[
          jax.ShapeDtypeStruct((N_NODES, HH), jnp.float32),
          jax.ShapeDtypeStruct((N_NODES, HH), jnp.float32),
          jax.ShapeDtypeStruct((N_NODES, HH), jnp.float32),
          jax.ShapeDtypeStruct((N_NODES, HH), jnp.float32),
          jax.ShapeDtypeStruct((NUM_GRAPHS, 2 * HIDDEN), jnp.float32),
      ],
  )(h0, h1, a0, a1, batch3, eps2, *wl, *wg)


def _m1_body(hl0, hl1, al0, al1, hg0, hg1, ag0, ag1, bt, eps_ref,
             w1l, b1l, w2l, b2l, w1g, b1g, w2g, b2g, pool):
  i = pl.program_id(0)
  oh = _onehot(bt[0, 0, :])

  pcs = []
  for (k, h0, h1, a0, a1, w1, b1, w2, b2) in (
      (0, hl0, hl1, al0, al1, w1l, b1l, w2l, b2l),
      (1, hg0, hg1, ag0, ag1, w1g, b1g, w2g, b2g)):
    h = jnp.concatenate([h0[...], h1[...]], axis=1)
    agg = jnp.concatenate([a0[...], a1[...]], axis=1)
    z = (1.0 + eps_ref[k]) * h + agg
    t = jnp.maximum(jnp.dot(z, w1[...], preferred_element_type=jnp.float32)
                    + b1[...], 0.0)
    z2 = jnp.dot(t, w2[...], preferred_element_type=jnp.float32) + b2[...]
    pcs.append(lax.dot_general(oh, z2, (((0,), (0,)), ((), ())),
                               precision=lax.Precision.HIGHEST,
                               preferred_element_type=jnp.float32))

  pc = jnp.concatenate(pcs, axis=1)

  @pl.when(i == 0)
  def _():
    pool[...] = pc

  @pl.when(i > 0)
  def _():
    pool[...] += pc


def _mlp1(hl0, hl1, al0, al1, hg0, hg1, ag0, ag1, batch3, eps2, wl, wg):
  wspec = [
      pl.BlockSpec((HIDDEN, HIDDEN), lambda i: (0, 0)),
      pl.BlockSpec((1, HIDDEN), lambda i: (0, 0)),
      pl.BlockSpec((HIDDEN, HIDDEN), lambda i: (0, 0)),
      pl.BlockSpec((1, HIDDEN), lambda i: (0, 0)),
  ]
  return pl.pallas_call(
      _m1_body,
      grid=(N_NODES // BM,),
      in_specs=[pl.BlockSpec((BM, HH), lambda i: (i, 0))] * 8 + [
          pl.BlockSpec((1, 1, BM), lambda i: (i, 0, 0)),
          pl.BlockSpec(memory_space=pltpu.SMEM),
      ] + wspec + wspec,
      out_specs=pl.BlockSpec((NUM_GRAPHS, 2 * HIDDEN), lambda i: (0, 0)),
      out_shape=jax.ShapeDtypeStruct((NUM_GRAPHS, 2 * HIDDEN), jnp.float32),
  )(hl0, hl1, al0, al1, hg0, hg1, ag0, ag1, batch3, eps2, *wl, *wg)


def _bn(x, g, b, eps=1e-5):
  mu = jnp.mean(x, axis=0, keepdims=True)
  var = jnp.mean((x - mu) * (x - mu), axis=0, keepdims=True)
  return (x - mu) / jnp.sqrt(var + eps) * g + b


def _head_body(pool0, pool1,
               w1g, b1g, g1g, be1g, w2g, b2g, g2g, be2g,
               w1l, b1l, g1l, be1l, w2l, b2l, g2l, be2l,
               wc, bc, out_ref, xl_ref, xg_ref):
  x_loc = jnp.concatenate([pool0[:, :HIDDEN], pool1[:, :HIDDEN]], axis=1)
  x_glob = jnp.concatenate([pool0[:, HIDDEN:], pool1[:, HIDDEN:]], axis=1)

  xg = jnp.dot(x_glob, w1g[...], preferred_element_type=jnp.float32) + b1g[...]
  xg = _bn(xg, g1g[...], be1g[...])
  xg = jnp.dot(xg, w2g[...], preferred_element_type=jnp.float32) + b2g[...]
  xg = _bn(xg, g2g[...], be2g[...])
  xg = jnp.maximum(xg, 0.0)

  xl = jnp.dot(x_loc, w1l[...], preferred_element_type=jnp.float32) + b1l[...]
  xl = _bn(xl, g1l[...], be1l[...])
  xl = jnp.dot(xl, w2l[...], preferred_element_type=jnp.float32) + b2l[...]
  xl = _bn(xl, g2l[...], be2l[...])
  xl = jnp.maximum(xl, 0.0)

  out_ref[...] = jnp.dot(xl + xg, wc[...],
                         preferred_element_type=jnp.float32) + bc[...]
  xl_ref[...] = xl
  xg_ref[...] = xg


def _head(pool0, pool1, hw):
  return pl.pallas_call(
      _head_body,
      in_specs=[pl.BlockSpec(a.shape, lambda: tuple(0 for _ in a.shape))
                for a in (pool0, pool1) + tuple(hw)],
      out_specs=[
          pl.BlockSpec((NUM_GRAPHS, 10), lambda: (0, 0)),
          pl.BlockSpec((NUM_GRAPHS, HIDDEN), lambda: (0, 0)),
          pl.BlockSpec((NUM_GRAPHS, HIDDEN), lambda: (0, 0)),
      ],
      out_shape=[
          jax.ShapeDtypeStruct((NUM_GRAPHS, 10), jnp.float32),
          jax.ShapeDtypeStruct((NUM_GRAPHS, HIDDEN), jnp.float32),
          jax.ShapeDtypeStruct((NUM_GRAPHS, HIDDEN), jnp.float32),
      ],
  )(pool0, pool1, *hw)


def kernel(x, edge_index, edge_attr, batch, params):
  p = params
  src = edge_index[0].astype(jnp.int32)
  dst = edge_index[1].astype(jnp.int32)
  batch3 = batch.astype(jnp.int32).reshape(N_NODES // BM, 1, BM)
  zrows = jnp.zeros((ROWS_A, HH), jnp.float32)

  def row(v):
    return v.reshape(1, -1)

  h0, h1 = _enc(x, p['W_enc'], row(p['b_enc']))
  ea0, ea1 = _ea_embed(edge_attr, p['W_emb'], row(p['b_emb']))

  a0, a1 = _sc_pass(h0, h1, ea0, ea1, src, dst, zrows)

  Ll0, Lg0 = p['gnn_loc'][0], p['gnn_glob'][0]
  Ll1, Lg1 = p['gnn_loc'][1], p['gnn_glob'][1]
  eps2 = jnp.stack([Ll0['eps'], Lg0['eps']])
  wl = (Ll0['W1'], row(Ll0['b1']), Ll0['W2'], row(Ll0['b2']))
  wg = (Lg0['W1'], row(Lg0['b1']), Lg0['W2'], row(Lg0['b2']))
  hl0, hl1, hg0, hg1, pool0 = _mlp0(
      h0, h1, a0, a1, batch3, eps2, wl, wg)

  al0, al1 = _sc_pass(hl0, hl1, ea0, ea1, src, dst, zrows)
  ag0, ag1 = _sc_pass(hg0, hg1, ea0, ea1, src, dst, zrows)

  eps2b = jnp.stack([Ll1['eps'], Lg1['eps']])
  pool1 = _mlp1(hl0, hl1, al0, al1, hg0, hg1, ag0, ag1, batch3, eps2b,
                (Ll1['W1'], row(Ll1['b1']), Ll1['W2'], row(Ll1['b2'])),
                (Lg1['W1'], row(Lg1['b1']), Lg1['W2'], row(Lg1['b2'])))

  hw = (p['W_out1_glob'], row(p['b_out1_glob']),
        row(p['bn1_g_glob']), row(p['bn1_b_glob']),
        p['W_out2_glob'], row(p['b_out2_glob']),
        row(p['bn2_g_glob']), row(p['bn2_b_glob']),
        p['W_out1_loc'], row(p['b_out1_loc']),
        row(p['bn1_g_loc']), row(p['bn1_b_loc']),
        p['W_out2_loc'], row(p['b_out2_loc']),
        row(p['bn2_g_loc']), row(p['bn2_b_loc']),
        p['W_clf'], row(p['b_clf']))

  out, xl, xg = _head(pool0, pool1, hw)
  return (out, xl, xg)
